# speedup vs baseline: 4.1670x; 4.1670x over previous
import functools

import jax
import jax.numpy as jnp
from jax import lax
from jax.experimental import pallas as pl
from jax.experimental.pallas import tpu as pltpu
from jax.experimental.pallas import tpu_sc as plsc

_N = 10000
_D = 128
_E = 320000
_G = 64
_OUT = 10

_NW = 32
_NPAD = 10240
_ZROWS = _NPAD // 16
_CHUNK = 128
_EPT = _E // _NW
_CH = -(-_EPT // _CHUNK)
_EPAD = _NW * _CH * _CHUNK

_mesh = plsc.VectorSubcoreMesh(core_axis_name="c", subcore_axis_name="s")


@functools.partial(
    pl.kernel,
    out_type=jax.ShapeDtypeStruct((2, _NPAD, _D), jnp.float32),
    mesh=_mesh,
    scratch_types=[
        pltpu.VMEM((_CH, _CHUNK), jnp.int32),
        pltpu.VMEM((_CH, _CHUNK), jnp.int32),
        pltpu.VMEM((_CHUNK, _D), jnp.float32),
        pltpu.VMEM_SHARED((_NPAD, _D), jnp.float32),
        pltpu.SemaphoreType.DMA,
    ],
)
def _sc_agg(x_hbm, src_hbm, dst_hbm, z_hbm, out_hbm, srcv, dstv, rows, agg_sh, sem):
    cid = lax.axis_index("c")
    sid = lax.axis_index("s")
    wid = sid * 2 + cid
    pltpu.sync_copy(z_hbm, agg_sh.at[pl.ds(sid * _ZROWS, _ZROWS)])
    pltpu.sync_copy(src_hbm.at[wid], srcv)
    pltpu.sync_copy(dst_hbm.at[wid], dstv)
    plsc.subcore_barrier()

    @pl.loop(0, _CH)
    def _edge_chunks(j):
        pltpu.async_copy(x_hbm.at[srcv.at[j]], rows, sem).wait()
        pltpu.sync_copy(rows, agg_sh.at[dstv.at[j]], add=True)

    plsc.subcore_barrier()
    pltpu.sync_copy(agg_sh.at[pl.ds(sid * _ZROWS, _ZROWS)],
                    out_hbm.at[cid, pl.ds(sid * _ZROWS, _ZROWS)])


def _layer_body(h_ref, agg_ref, w1_ref, b1_ref, w2_ref, b2_ref, g_ref, be_ref, o_ref):
    u = h_ref[...] + agg_ref[0] + agg_ref[1]
    t = jnp.dot(u, w1_ref[...], preferred_element_type=jnp.float32) + b1_ref[...]
    t = jnp.where(t > 0, t, 0.01 * t)
    v = jnp.dot(t, w2_ref[...], preferred_element_type=jnp.float32) + b2_ref[...]
    v = jnp.where(v > 0, v, 0.01 * v)
    m = jnp.mean(v, axis=0, keepdims=True)
    c = v - m
    var = jnp.mean(c * c, axis=0, keepdims=True)
    o_ref[...] = c * lax.rsqrt(var + 1e-5) * g_ref[...] + be_ref[...]


_layer_call = pl.pallas_call(
    _layer_body,
    out_shape=jax.ShapeDtypeStruct((_N, _D), jnp.float32),
)


def _head_body(h_ref, batch_ref, wh1_ref, bh1_ref, wh2_ref, bh2_ref, o_ref):
    onehot = (batch_ref[...] == lax.broadcasted_iota(jnp.int32, (1, _G), 1)
              ).astype(jnp.float32)
    pooled = lax.dot_general(onehot, h_ref[...], (((0,), (0,)), ((), ())),
                             preferred_element_type=jnp.float32)
    counts = jnp.sum(onehot, axis=0)[:, None]
    pooled = pooled / jnp.maximum(counts, 1.0)
    t = jnp.dot(pooled, wh1_ref[...], preferred_element_type=jnp.float32) + bh1_ref[...]
    t = jnp.where(t > 0, t, 0.01 * t)
    o_ref[...] = jnp.dot(t, wh2_ref[...], preferred_element_type=jnp.float32) + bh2_ref[...]


_head_call = pl.pallas_call(
    _head_body,
    out_shape=jax.ShapeDtypeStruct((_G, _OUT), jnp.float32),
)


def kernel(x, edge_index, batch,
           W1_0, b1_0, W2_0, b2_0, g_0, be_0,
           W1_1, b1_1, W2_1, b2_1, g_1, be_1,
           W1_2, b1_2, W2_2, b2_2, g_2, be_2,
           Wh1, bh1, Wh2, bh2):
    src = edge_index[0]
    dst = edge_index[1]
    pad = _EPAD - _E
    srcp = jnp.concatenate([src, jnp.zeros((pad,), jnp.int32)]).reshape(_NW, _CH, _CHUNK)
    dstp = jnp.concatenate([dst, jnp.full((pad,), _NPAD - 1, jnp.int32)]).reshape(_NW, _CH, _CHUNK)
    zeros = jnp.zeros((_ZROWS, _D), jnp.float32)

    layers = [
        (W1_0, b1_0, W2_0, b2_0, g_0, be_0),
        (W1_1, b1_1, W2_1, b2_1, g_1, be_1),
        (W1_2, b1_2, W2_2, b2_2, g_2, be_2),
    ]
    h = x
    for (W1, b1, W2, b2, g, be) in layers:
        agg = _sc_agg(h, srcp, dstp, zeros)[:, :_N, :]
        h = _layer_call(h, agg, W1, b1.reshape(1, _D), W2, b2.reshape(1, _D),
                        g.reshape(1, _D), be.reshape(1, _D))
    return _head_call(h, batch.reshape(_N, 1), Wh1, bh1.reshape(1, _D),
                      Wh2, bh2.reshape(1, _OUT))
